# Initial kernel scaffold; baseline (speedup 1.0000x reference)
import jax
import jax.numpy as jnp
from jax.experimental import pallas as pl


def kernel(table1, table2, word, nbor, negs):
    raise NotImplementedError("write your pallas kernel here")


# speedup vs baseline: 1.8807x; 1.8807x over previous
import functools

import jax
import jax.numpy as jnp
from jax import lax
from jax.experimental import pallas as pl
from jax.experimental.pallas import tpu as pltpu
from jax.experimental.pallas import tpu_sc as plsc

NW = 32
CH = 128


def _sc_gather_body(t1, t2, widx, nidx, cidx, a_out, b_out, c_out,
                    idx_v, buf0, buf1, gsem0, gsem1):
    wid = lax.axis_index("s") * 2 + lax.axis_index("c")
    nw_chunks = widx.shape[1]
    nn_chunks = nidx.shape[1]
    nc_chunks = cidx.shape[1]

    def run_segment(idx_hbm, n_chunks, table, out):
        pltpu.sync_copy(idx_hbm.at[wid], idx_v.at[pl.ds(0, n_chunks)])
        base = wid * n_chunks * CH

        def body(i, _):
            j0 = 2 * i
            j1 = 2 * i + 1
            h0 = pltpu.async_copy(table.at[idx_v.at[j0]], buf0, gsem0)
            h1 = pltpu.async_copy(table.at[idx_v.at[j1]], buf1, gsem1)
            h0.wait()
            pltpu.sync_copy(buf0, out.at[pl.ds(base + j0 * CH, CH)])
            h1.wait()
            pltpu.sync_copy(buf1, out.at[pl.ds(base + j1 * CH, CH)])
            return 0

        lax.fori_loop(0, n_chunks // 2, body, 0)

    run_segment(widx, nw_chunks, t1, a_out)
    run_segment(nidx, nn_chunks, t2, b_out)
    run_segment(cidx, nc_chunks, t2, c_out)


def _sc_gather(table1, table2, widx, nidx, cidx):
    E = table1.shape[1]
    nb = widx.shape[0] * widx.shape[1] * widx.shape[2]
    nn = nidx.shape[0] * nidx.shape[1] * nidx.shape[2]
    nc = cidx.shape[0] * cidx.shape[1] * cidx.shape[2]
    max_chunks = max(widx.shape[1], nidx.shape[1], cidx.shape[1])
    mesh = plsc.VectorSubcoreMesh(core_axis_name="c", subcore_axis_name="s")
    fn = functools.partial(
        pl.kernel,
        mesh=mesh,
        out_type=[
            jax.ShapeDtypeStruct((nb, E), jnp.float32),
            jax.ShapeDtypeStruct((nn, E), jnp.float32),
            jax.ShapeDtypeStruct((nc, E), jnp.float32),
        ],
        scratch_types=[
            pltpu.VMEM((max_chunks, CH), jnp.int32),
            pltpu.VMEM((CH, E), jnp.float32),
            pltpu.VMEM((CH, E), jnp.float32),
            pltpu.SemaphoreType.DMA,
            pltpu.SemaphoreType.DMA,
        ],
        compiler_params=pltpu.CompilerParams(use_tc_tiling_on_sc=False),
    )(_sc_gather_body)
    return fn(table1, table2, widx, nidx, cidx)


def _tc_finish_body(a_ref, b_ref, c_ref, a_out, loss_ref):
    a = a_ref[...]
    n2 = jnp.sum(a * a, axis=1, keepdims=True)
    n = jnp.sqrt(n2)
    scale = jnp.where(n > 1.0, 1.0 / jnp.maximum(n, 1e-12), 1.0)
    a_n = a * scale
    a_out[...] = a_n

    a3 = a_n[:, None, :]
    sim1 = jnp.sum(b_ref[...] * a3, axis=2)
    sim2 = jnp.sum(c_ref[...] * a3, axis=2)
    p1 = jnp.sum(jnp.log(jax.nn.sigmoid(sim1 + 0.01)))
    p2 = jnp.sum(jnp.log(jax.nn.sigmoid(-sim2 + 0.01)))
    partial = -(p1 + p2)

    @pl.when(pl.program_id(0) == 0)
    def _():
        loss_ref[...] = jnp.zeros_like(loss_ref)

    loss_ref[...] += jnp.full((1, 1), partial, jnp.float32)


def _tc_finish(a_raw, embed_b, embed_c):
    B, E = a_raw.shape
    W = embed_b.shape[1]
    NEG = embed_c.shape[1]
    bs = 256
    grid = B // bs
    return pl.pallas_call(
        _tc_finish_body,
        grid=(grid,),
        in_specs=[
            pl.BlockSpec((bs, E), lambda i: (i, 0)),
            pl.BlockSpec((bs, W, E), lambda i: (i, 0, 0)),
            pl.BlockSpec((bs, NEG, E), lambda i: (i, 0, 0)),
        ],
        out_specs=[
            pl.BlockSpec((bs, E), lambda i: (i, 0)),
            pl.BlockSpec((1, 1), lambda i: (0, 0)),
        ],
        out_shape=[
            jax.ShapeDtypeStruct((B, E), jnp.float32),
            jax.ShapeDtypeStruct((1, 1), jnp.float32),
        ],
    )(a_raw, embed_b, embed_c)


def kernel(table1, table2, word, nbor, negs):
    B = word.shape[0]
    E = table1.shape[1]
    W = nbor.shape[1]
    NEG = negs.shape[1]

    word_i = word.astype(jnp.int32).reshape(NW, -1, CH)
    nbor_i = nbor.astype(jnp.int32).reshape(NW, -1, CH)
    negs_i = negs.astype(jnp.int32).reshape(NW, -1, CH)

    a_raw, b_flat, c_flat = _sc_gather(table1, table2, word_i, nbor_i, negs_i)
    embed_b = b_flat.reshape(B, W, E)
    embed_c = c_flat.reshape(B, NEG, E)

    embed_a, loss2 = _tc_finish(a_raw, embed_b, embed_c)
    loss = loss2.reshape(())
    return (loss, embed_a, embed_b, embed_c)


# speedup vs baseline: 5.3128x; 2.8249x over previous
import functools

import jax
import jax.numpy as jnp
from jax import lax
from jax.experimental import pallas as pl
from jax.experimental.pallas import tpu as pltpu
from jax.experimental.pallas import tpu_sc as plsc

NW = 32
CH = 128


def _run_segment(wid, idx_hbm, table, out, idx_v, buf0, buf1, gsem0, gsem1):
    n_chunks = idx_hbm.shape[1]
    pltpu.sync_copy(idx_hbm.at[wid], idx_v.at[pl.ds(0, n_chunks)])
    base = wid * n_chunks * CH

    def body(i, _):
        j0 = 2 * i
        j1 = 2 * i + 1
        h0 = pltpu.async_copy(table.at[idx_v.at[j0]], buf0, gsem0)
        h1 = pltpu.async_copy(table.at[idx_v.at[j1]], buf1, gsem1)
        h0.wait()
        pltpu.sync_copy(buf0, out.at[pl.ds(base + j0 * CH, CH)])
        h1.wait()
        pltpu.sync_copy(buf1, out.at[pl.ds(base + j1 * CH, CH)])
        return 0

    lax.fori_loop(0, n_chunks // 2, body, 0)


def _sc_gather_ab_body(t1, t2, widx, nidx, a_out, b_out,
                       idx_v, buf0, buf1, gsem0, gsem1):
    wid = lax.axis_index("s") * 2 + lax.axis_index("c")
    _run_segment(wid, widx, t1, a_out, idx_v, buf0, buf1, gsem0, gsem1)
    _run_segment(wid, nidx, t2, b_out, idx_v, buf0, buf1, gsem0, gsem1)


def _sc_gather_c_body(t2, cidx, c_out, idx_v, buf0, buf1, gsem0, gsem1):
    wid = lax.axis_index("s") * 2 + lax.axis_index("c")
    _run_segment(wid, cidx, t2, c_out, idx_v, buf0, buf1, gsem0, gsem1)


def _sc_scratch(max_chunks, E):
    return [
        pltpu.VMEM((max_chunks, CH), jnp.int32),
        pltpu.VMEM((CH, E), jnp.float32),
        pltpu.VMEM((CH, E), jnp.float32),
        pltpu.SemaphoreType.DMA,
        pltpu.SemaphoreType.DMA,
    ]


_MESH = dict(core_axis_name="c", subcore_axis_name="s")


def _sc_gather_ab(table1, table2, widx, nidx):
    E = table1.shape[1]
    nb = widx.shape[0] * widx.shape[1] * widx.shape[2]
    nn = nidx.shape[0] * nidx.shape[1] * nidx.shape[2]
    fn = functools.partial(
        pl.kernel,
        mesh=plsc.VectorSubcoreMesh(**_MESH),
        out_type=[
            jax.ShapeDtypeStruct((nb, E), jnp.float32),
            jax.ShapeDtypeStruct((nn, E), jnp.float32),
        ],
        scratch_types=_sc_scratch(max(widx.shape[1], nidx.shape[1]), E),
        compiler_params=pltpu.CompilerParams(use_tc_tiling_on_sc=False),
    )(_sc_gather_ab_body)
    return fn(table1, table2, widx, nidx)


def _sc_gather_c(table2, cidx):
    E = table2.shape[1]
    nc = cidx.shape[0] * cidx.shape[1] * cidx.shape[2]
    fn = functools.partial(
        pl.kernel,
        mesh=plsc.VectorSubcoreMesh(**_MESH),
        out_type=jax.ShapeDtypeStruct((nc, E), jnp.float32),
        scratch_types=_sc_scratch(cidx.shape[1], E),
        compiler_params=pltpu.CompilerParams(use_tc_tiling_on_sc=False),
    )(_sc_gather_c_body)
    return fn(table2, cidx)


def _tc_renorm_body(a_ref, at_out):
    at_raw = a_ref[...].T
    n2 = jnp.sum(at_raw * at_raw, axis=0, keepdims=True)
    n = jnp.sqrt(n2)
    scale = jnp.where(n > 1.0, 1.0 / jnp.maximum(n, 1e-12), 1.0)
    at_out[...] = at_raw * scale


def _tc_renorm(a_raw):
    B, E = a_raw.shape
    return pl.pallas_call(
        _tc_renorm_body,
        out_shape=jax.ShapeDtypeStruct((E, B), jnp.float32),
    )(a_raw)


def _make_sim_body(sign):
    def body(at_ref, x_ref, xt_out, loss_ref):
        at = at_ref[...]
        xt = jnp.transpose(x_ref[...], (0, 2, 1))
        xt_out[...] = xt
        sim = jnp.sum(xt[0] * at, axis=0, keepdims=True)
        p = jnp.sum(jnp.log1p(jnp.exp(sign * sim - 0.01)))

        @pl.when(pl.program_id(0) == 0)
        def _():
            loss_ref[...] = jnp.zeros_like(loss_ref)

        loss_ref[...] += jnp.full((1, 1), p, jnp.float32)

    return body


_sim_body_b = _make_sim_body(-1.0)
_sim_body_c = _make_sim_body(1.0)


def _tc_sim(at, x3, body):
    E, B = at.shape
    K = x3.shape[0]
    return pl.pallas_call(
        body,
        grid=(K,),
        in_specs=[
            pl.BlockSpec((E, B), lambda i: (0, 0)),
            pl.BlockSpec((1, B, E), lambda i: (i, 0, 0)),
        ],
        out_specs=[
            pl.BlockSpec((1, E, B), lambda i: (i, 0, 0)),
            pl.BlockSpec((1, 1), lambda i: (0, 0)),
        ],
        out_shape=[
            jax.ShapeDtypeStruct((K, E, B), jnp.float32),
            jax.ShapeDtypeStruct((1, 1), jnp.float32),
        ],
    )(at, x3)


def kernel(table1, table2, word, nbor, negs):
    B = word.shape[0]
    E = table1.shape[1]
    W = nbor.shape[1]
    NEG = negs.shape[1]

    word_i = word.astype(jnp.int32).reshape(NW, -1, CH)
    nbor_i = nbor.astype(jnp.int32).T.reshape(NW, -1, CH)
    negs_i = negs.astype(jnp.int32).T.reshape(NW, -1, CH)

    a_raw, b_flat = _sc_gather_ab(table1, table2, word_i, nbor_i)
    c_flat = _sc_gather_c(table2, negs_i)
    b3 = b_flat.reshape(W, B, E)
    c3 = c_flat.reshape(NEG, B, E)

    at = _tc_renorm(a_raw)
    bt, loss_b = _tc_sim(at, b3, _sim_body_b)
    ct, loss_c = _tc_sim(at, c3, _sim_body_c)

    embed_a = at.T
    embed_b = jnp.transpose(bt, (2, 0, 1))
    embed_c = jnp.transpose(ct, (2, 0, 1))
    loss = (loss_b + loss_c).reshape(())
    return (loss, embed_a, embed_b, embed_c)


# speedup vs baseline: 6.4040x; 1.2054x over previous
import functools

import jax
import jax.numpy as jnp
from jax import lax
from jax.experimental import pallas as pl
from jax.experimental.pallas import tpu as pltpu
from jax.experimental.pallas import tpu_sc as plsc

NW = 32
CH = 128


def _run_segment(wid, idx_hbm, table, out, idx_v, buf0, buf1, gsem0, gsem1):
    n_chunks = idx_hbm.shape[1]
    pltpu.sync_copy(idx_hbm.at[wid], idx_v.at[pl.ds(0, n_chunks)])
    base = wid * n_chunks * CH

    def body(i, _):
        j0 = 2 * i
        j1 = 2 * i + 1
        h0 = pltpu.async_copy(table.at[idx_v.at[j0]], buf0, gsem0)
        h1 = pltpu.async_copy(table.at[idx_v.at[j1]], buf1, gsem1)
        h0.wait()
        pltpu.sync_copy(buf0, out.at[pl.ds(base + j0 * CH, CH)])
        h1.wait()
        pltpu.sync_copy(buf1, out.at[pl.ds(base + j1 * CH, CH)])
        return 0

    lax.fori_loop(0, n_chunks // 2, body, 0)


def _sc_gather_ab_body(t1, t2, widx, nidx, a_out, b_out,
                       idx_v, buf0, buf1, gsem0, gsem1):
    wid = lax.axis_index("s") * 2 + lax.axis_index("c")
    _run_segment(wid, widx, t1, a_out, idx_v, buf0, buf1, gsem0, gsem1)
    _run_segment(wid, nidx, t2, b_out, idx_v, buf0, buf1, gsem0, gsem1)


def _sc_gather_c_body(t2, cidx, c_out, idx_v, buf0, buf1, gsem0, gsem1):
    wid = lax.axis_index("s") * 2 + lax.axis_index("c")
    _run_segment(wid, cidx, t2, c_out, idx_v, buf0, buf1, gsem0, gsem1)


def _sc_scratch(max_chunks, E):
    return [
        pltpu.VMEM((max_chunks, CH), jnp.int32),
        pltpu.VMEM((CH, E), jnp.float32),
        pltpu.VMEM((CH, E), jnp.float32),
        pltpu.SemaphoreType.DMA,
        pltpu.SemaphoreType.DMA,
    ]


_MESH = dict(core_axis_name="c", subcore_axis_name="s")


def _sc_gather_ab(table1, table2, widx, nidx):
    E = table1.shape[1]
    nb = widx.shape[0] * widx.shape[1] * widx.shape[2]
    nn = nidx.shape[0] * nidx.shape[1] * nidx.shape[2]
    fn = functools.partial(
        pl.kernel,
        mesh=plsc.VectorSubcoreMesh(**_MESH),
        out_type=[
            jax.ShapeDtypeStruct((nb, E), jnp.float32),
            jax.ShapeDtypeStruct((nn, E), jnp.float32),
        ],
        scratch_types=_sc_scratch(max(widx.shape[1], nidx.shape[1]), E),
        compiler_params=pltpu.CompilerParams(use_tc_tiling_on_sc=False),
    )(_sc_gather_ab_body)
    return fn(table1, table2, widx, nidx)


def _sc_gather_c(table2, cidx):
    E = table2.shape[1]
    nc = cidx.shape[0] * cidx.shape[1] * cidx.shape[2]
    fn = functools.partial(
        pl.kernel,
        mesh=plsc.VectorSubcoreMesh(**_MESH),
        out_type=jax.ShapeDtypeStruct((nc, E), jnp.float32),
        scratch_types=_sc_scratch(cidx.shape[1], E),
        compiler_params=pltpu.CompilerParams(use_tc_tiling_on_sc=False),
    )(_sc_gather_c_body)
    return fn(table2, cidx)


def _tc_renorm_body(a_ref, at_out):
    at_raw = a_ref[...].T
    n2 = jnp.sum(at_raw * at_raw, axis=0, keepdims=True)
    n = jnp.sqrt(n2)
    scale = jnp.where(n > 1.0, 1.0 / jnp.maximum(n, 1e-12), 1.0)
    at_out[...] = at_raw * scale


def _tc_renorm(a_raw):
    B, E = a_raw.shape
    return pl.pallas_call(
        _tc_renorm_body,
        out_shape=jax.ShapeDtypeStruct((E, B), jnp.float32),
    )(a_raw)


def _make_sim_body(sign):
    def body(at_ref, x_ref, xt_out, loss_ref):
        at = at_ref[...]
        E, B = at.shape
        H = B // 2
        xt = x_ref[0].T
        lo = xt[:E]
        hi = xt[E:]
        xt_out[0, :, :H] = lo
        xt_out[0, :, H:] = hi
        sim_lo = jnp.sum(lo * at[:, :H], axis=0, keepdims=True)
        sim_hi = jnp.sum(hi * at[:, H:], axis=0, keepdims=True)
        p = (jnp.sum(jnp.log1p(jnp.exp(sign * sim_lo - 0.01)))
             + jnp.sum(jnp.log1p(jnp.exp(sign * sim_hi - 0.01))))

        @pl.when(pl.program_id(0) == 0)
        def _():
            loss_ref[...] = jnp.zeros_like(loss_ref)

        loss_ref[...] += jnp.full((1, 1), p, jnp.float32)

    return body


_sim_body_b = _make_sim_body(-1.0)
_sim_body_c = _make_sim_body(1.0)


def _tc_sim(at, x3, body):
    E, B = at.shape
    K = x3.shape[0]
    return pl.pallas_call(
        body,
        grid=(K,),
        in_specs=[
            pl.BlockSpec((E, B), lambda i: (0, 0)),
            pl.BlockSpec((1, B // 2, 2 * E), lambda i: (i, 0, 0)),
        ],
        out_specs=[
            pl.BlockSpec((1, E, B), lambda i: (i, 0, 0)),
            pl.BlockSpec((1, 1), lambda i: (0, 0)),
        ],
        out_shape=[
            jax.ShapeDtypeStruct((K, E, B), jnp.float32),
            jax.ShapeDtypeStruct((1, 1), jnp.float32),
        ],
    )(at, x3)


def kernel(table1, table2, word, nbor, negs):
    B = word.shape[0]
    E = table1.shape[1]
    W = nbor.shape[1]
    NEG = negs.shape[1]

    word_i = word.astype(jnp.int32).reshape(NW, -1, CH)
    H = B // 2

    def _perm(x):
        xt = x.astype(jnp.int32).T
        return jnp.transpose(xt.reshape(-1, 2, H), (0, 2, 1)).reshape(NW, -1, CH)

    nbor_i = _perm(nbor)
    negs_i = _perm(negs)

    a_raw, b_flat = _sc_gather_ab(table1, table2, word_i, nbor_i)
    c_flat = _sc_gather_c(table2, negs_i)
    b3 = b_flat.reshape(W, H, 2 * E)
    c3 = c_flat.reshape(NEG, H, 2 * E)

    at = _tc_renorm(a_raw)
    bt, loss_b = _tc_sim(at, b3, _sim_body_b)
    ct, loss_c = _tc_sim(at, c3, _sim_body_c)

    embed_a = at.T
    embed_b = jnp.transpose(bt, (2, 0, 1))
    embed_c = jnp.transpose(ct, (2, 0, 1))
    loss = (loss_b + loss_c).reshape(())
    return (loss, embed_a, embed_b, embed_c)
